# baseline (speedup 1.0000x reference)
import functools

import numpy as np
import jax
import jax.numpy as jnp
from jax import lax
from jax.experimental import pallas as pl
from jax.experimental.pallas import tpu as pltpu
from jax.experimental.pallas import tpu_sc as plsc

_SHAPE = (16, 16, 16)
_PROB = 0.5
_B = 4
_D = 1024
_N = _SHAPE[0] * _SHAPE[1] * _SHAPE[2]
_K = int(round(_N * (1.0 - _PROB)))

_RUN = 8
_NRUNS = _K // _RUN
_NW = 32
_RUNS_PER_W = (_B * _NRUNS) // _NW


def _keep_rows(batch, shape, prob):
    rng = np.random.RandomState(0)
    X, Y, Z = shape
    N = X * Y * Z
    wanted = int(round(N * (1.0 - prob)))
    tokens_per_dim = [Y * Z, X * Z, X * Y]
    full_slices = [wanted // t for t in tokens_per_dim]
    remainders = [wanted % t for t in tokens_per_dim]
    out = []
    for _ in range(batch):
        dim = int(rng.randint(0, 3))
        n_keep = full_slices[dim]
        leftover = remainders[dim]
        start = int(rng.randint(1, shape[dim] - (n_keep + 1) + 1))
        prev = int(rng.randint(0, 2))
        next_slice = start - 1 if prev else start + n_keep
        mask = np.zeros(shape, dtype=bool)
        if dim == 0:
            mask[start:start + n_keep, :, :] = True
            if leftover > 0:
                perm = rng.permutation(Y * Z)[:leftover]
                mask[next_slice, perm // Z, perm % Z] = True
        elif dim == 1:
            mask[:, start:start + n_keep, :] = True
            if leftover > 0:
                perm = rng.permutation(X * Z)[:leftover]
                mask[perm // Z, next_slice, perm % Z] = True
        else:
            mask[:, :, start:start + n_keep] = True
            if leftover > 0:
                perm = rng.permutation(X * Y)[:leftover]
                mask[perm // Y, perm % Y, next_slice] = True
        out.append(np.nonzero(mask.reshape(-1))[0])
    return np.stack(out, axis=0).astype(np.int32)


_KEEP = _keep_rows(_B, _SHAPE, _PROB)
assert _KEEP.shape == (_B, _K)

_grp = _KEEP.reshape(_B, _NRUNS, _RUN).astype(np.int64)
assert (_grp == _grp[:, :, :1] + np.arange(_RUN)).all(), "non-contiguous run"
_R0 = _grp[:, :, 0]
_A = _R0[:, 0]
_Q = _R0[:, 1] - _R0[:, 0]
_P = _R0[:, 16] - _R0[:, 0]
_j = np.arange(_NRUNS)
assert (
    _A[:, None] + (_j // 16)[None, :] * _P[:, None]
    + (_j % 16)[None, :] * _Q[:, None] == _R0
).all(), "affine run-start fit failed"
_A, _P, _Q = _A.tolist(), _P.tolist(), _Q.tolist()


_GIDX = (_KEEP.astype(np.int64) + np.arange(_B)[:, None] * _N).reshape(-1)
_GIDX = _GIDX.astype(np.int32)

_ROWS_PER_W = (_B * _K) // _NW
_CH = 32
_NCH = _ROWS_PER_W // _CH
_NBUF = 3


def _sel4(b, vals):
    v = [jnp.int32(int(x)) for x in vals]
    return jnp.where(
        b == 0, v[0], jnp.where(b == 1, v[1], jnp.where(b == 2, v[2], v[3]))
    )


def _gather_body(x_hbm, o_hbm, idx_v, buf, *sems):
    gsems = sems[:_NBUF]
    ssems = sems[_NBUF:]
    c = lax.axis_index("c")
    s = lax.axis_index("s")
    w = s * 2 + c
    base = w * _ROWS_PER_W
    b = w // 8
    a_c = _sel4(b, _A)
    p_c = _sel4(b, _P)
    q_c = _sel4(b, _Q)
    boff = b * _N
    lane = lax.iota(jnp.int32, 16)
    for i in range(_NCH):
        for v in range(_CH // 16):
            k = base + i * _CH + v * 16 + lane
            rem = jnp.bitwise_and(k, _K - 1)
            j = jnp.right_shift(rem, 3)
            t = jnp.bitwise_and(rem, 7)
            src = (
                boff + a_c + jnp.right_shift(j, 4) * p_c
                + jnp.bitwise_and(j, 15) * q_c + t
            )
            idx_v[pl.ds(i * _CH + v * 16, 16)] = src

    def gather(i):
        return pltpu.make_async_copy(
            x_hbm.at[idx_v.at[pl.ds(i * _CH, _CH)]],
            buf.at[i % _NBUF],
            gsems[i % _NBUF],
        )

    def put(i):
        return pltpu.make_async_copy(
            buf.at[i % _NBUF],
            o_hbm.at[pl.ds(base + i * _CH, _CH), :],
            ssems[i % _NBUF],
        )

    g_st = {}
    s_st = {}
    for i in range(min(_NBUF, _NCH)):
        g_st[i] = gather(i)
        g_st[i].start()
    last_waited = -1
    for i in range(_NCH):
        la = i + _NBUF - 1
        if i >= 1 and la < _NCH:
            s_st[i - 1].wait()
            last_waited = i - 1
            g_st[la] = gather(la)
            g_st[la].start()
        g_st[i].wait()
        s_st[i] = put(i)
        s_st[i].start()
    for i in range(last_waited + 1, _NCH):
        s_st[i].wait()


def kernel(x):
    x2 = x.reshape(_B * _N, _D)
    mesh = plsc.VectorSubcoreMesh(core_axis_name="c", subcore_axis_name="s")
    run = functools.partial(
        pl.kernel,
        out_type=jax.ShapeDtypeStruct((_B * _K, _D), jnp.float32),
        mesh=mesh,
        scratch_types=[
            pltpu.VMEM((_NCH * _CH,), jnp.int32),
            pltpu.VMEM((_NBUF, _CH, _D), jnp.float32),
        ] + [pltpu.SemaphoreType.DMA] * (2 * _NBUF),
    )(_gather_body)
    out = run(x2)
    return out.reshape(_B, _K, _D)


# speedup vs baseline: 1.0073x; 1.0073x over previous
import functools

import numpy as np
import jax
import jax.numpy as jnp
from jax import lax
from jax.experimental import pallas as pl
from jax.experimental.pallas import tpu as pltpu
from jax.experimental.pallas import tpu_sc as plsc

_SHAPE = (16, 16, 16)
_PROB = 0.5
_B = 4
_D = 1024
_N = _SHAPE[0] * _SHAPE[1] * _SHAPE[2]
_K = int(round(_N * (1.0 - _PROB)))

_RUN = 8
_NRUNS = _K // _RUN
_NW = 32
_RUNS_PER_W = (_B * _NRUNS) // _NW


def _keep_rows(batch, shape, prob):
    rng = np.random.RandomState(0)
    X, Y, Z = shape
    N = X * Y * Z
    wanted = int(round(N * (1.0 - prob)))
    tokens_per_dim = [Y * Z, X * Z, X * Y]
    full_slices = [wanted // t for t in tokens_per_dim]
    remainders = [wanted % t for t in tokens_per_dim]
    out = []
    for _ in range(batch):
        dim = int(rng.randint(0, 3))
        n_keep = full_slices[dim]
        leftover = remainders[dim]
        start = int(rng.randint(1, shape[dim] - (n_keep + 1) + 1))
        prev = int(rng.randint(0, 2))
        next_slice = start - 1 if prev else start + n_keep
        mask = np.zeros(shape, dtype=bool)
        if dim == 0:
            mask[start:start + n_keep, :, :] = True
            if leftover > 0:
                perm = rng.permutation(Y * Z)[:leftover]
                mask[next_slice, perm // Z, perm % Z] = True
        elif dim == 1:
            mask[:, start:start + n_keep, :] = True
            if leftover > 0:
                perm = rng.permutation(X * Z)[:leftover]
                mask[perm // Z, next_slice, perm % Z] = True
        else:
            mask[:, :, start:start + n_keep] = True
            if leftover > 0:
                perm = rng.permutation(X * Y)[:leftover]
                mask[perm // Y, perm % Y, next_slice] = True
        out.append(np.nonzero(mask.reshape(-1))[0])
    return np.stack(out, axis=0).astype(np.int32)


_KEEP = _keep_rows(_B, _SHAPE, _PROB)
assert _KEEP.shape == (_B, _K)

_grp = _KEEP.reshape(_B, _NRUNS, _RUN).astype(np.int64)
assert (_grp == _grp[:, :, :1] + np.arange(_RUN)).all(), "non-contiguous run"
_R0 = _grp[:, :, 0]
_A = _R0[:, 0]
_Q = _R0[:, 1] - _R0[:, 0]
_P = _R0[:, 16] - _R0[:, 0]
_j = np.arange(_NRUNS)
assert (
    _A[:, None] + (_j // 16)[None, :] * _P[:, None]
    + (_j % 16)[None, :] * _Q[:, None] == _R0
).all(), "affine run-start fit failed"
_A, _P, _Q = _A.tolist(), _P.tolist(), _Q.tolist()


_ROWS_PER_W = (_B * _K) // _NW
_CH = 32
_NCH = _ROWS_PER_W // _CH
_NBUF = 3


def _sel4(b, vals):
    v = [jnp.int32(int(x)) for x in vals]
    return jnp.where(
        b == 0, v[0], jnp.where(b == 1, v[1], jnp.where(b == 2, v[2], v[3]))
    )


def _gather_body(x_hbm, o_hbm, idx_v, buf, *sems):
    gsems = sems[:_NBUF]
    ssems = sems[_NBUF:]
    c = lax.axis_index("c")
    s = lax.axis_index("s")
    w = s * 2 + c
    base = w * _ROWS_PER_W
    b = w // 8
    a_c = _sel4(b, _A)
    p_c = _sel4(b, _P)
    q_c = _sel4(b, _Q)
    boff = b * _N
    lane = lax.iota(jnp.int32, 16)
    for i in range(_NCH):
        for v in range(_CH // 16):
            k = base + i * _CH + v * 16 + lane
            rem = jnp.bitwise_and(k, _K - 1)
            j = jnp.right_shift(rem, 3)
            t = jnp.bitwise_and(rem, 7)
            src = (
                boff + a_c + jnp.right_shift(j, 4) * p_c
                + jnp.bitwise_and(j, 15) * q_c + t
            )
            idx_v[pl.ds(i * _CH + v * 16, 16)] = src

    def gather(i):
        return pltpu.make_async_copy(
            x_hbm.at[idx_v.at[pl.ds(i * _CH, _CH)]],
            buf.at[i % _NBUF],
            gsems[i % _NBUF],
        )

    def put(i):
        return pltpu.make_async_copy(
            buf.at[i % _NBUF],
            o_hbm.at[pl.ds(base + i * _CH, _CH), :],
            ssems[i % _NBUF],
        )

    g_st = {}
    s_st = {}
    for i in range(min(_NBUF, _NCH)):
        g_st[i] = gather(i)
        g_st[i].start()
    last_waited = -1
    for i in range(_NCH):
        la = i + _NBUF - 1
        if i >= 1 and la < _NCH:
            s_st[i - 1].wait()
            last_waited = i - 1
            g_st[la] = gather(la)
            g_st[la].start()
        g_st[i].wait()
        s_st[i] = put(i)
        s_st[i].start()
    for i in range(last_waited + 1, _NCH):
        s_st[i].wait()


def kernel(x):
    x2 = x.reshape(_B * _N, _D)
    mesh = plsc.VectorSubcoreMesh(core_axis_name="c", subcore_axis_name="s")
    run = functools.partial(
        pl.kernel,
        out_type=jax.ShapeDtypeStruct((_B * _K, _D), jnp.float32),
        mesh=mesh,
        scratch_types=[
            pltpu.VMEM((_NCH * _CH,), jnp.int32),
            pltpu.VMEM((_NBUF, _CH, _D), jnp.float32),
        ] + [pltpu.SemaphoreType.DMA] * (2 * _NBUF),
    )(_gather_body)
    out = run(x2)
    return out.reshape(_B, _K, _D)
